# Initial kernel scaffold; baseline (speedup 1.0000x reference)
import jax
import jax.numpy as jnp
from jax.experimental import pallas as pl


def kernel(x, W, b):
    raise NotImplementedError("write your pallas kernel here")


# speedup vs baseline: 113.4003x; 113.4003x over previous
import functools

import jax
import jax.numpy as jnp
from jax import lax
from jax.experimental import pallas as pl
from jax.experimental.pallas import tpu as pltpu
from jax.experimental.pallas import tpu_sc as plsc

_K = 5
_T = 256
_NROWS = 256
_NC = 2
_NS = 16
_NW = _NC * _NS
_ROWS_PER = _NROWS // _NW
_PAD = 8
_XPAD = 272
_LANES = 16


def _round_bf16(v):
    u = plsc.bitcast(v, jnp.int32)
    lsb = jax.lax.shift_right_logical(u, 16) & 1
    u = (u + 0x7FFF + lsb) & jnp.int32(-65536)
    return plsc.bitcast(u, jnp.float32)


def _sc_body(x_hbm, wb_hbm, out_hbm, xpad_v, out_v, wb_v):
    wid = lax.axis_index("s") * _NC + lax.axis_index("c")
    pltpu.sync_copy(wb_hbm, wb_v)
    iota = lax.iota(jnp.int32, _LANES)
    iotaf = iota.astype(jnp.float32)
    wv = [[_round_bf16(
        plsc.load_gather(wb_v, [jnp.full((_LANES,), 1 + k * _K + j, jnp.int32)]))
           for j in range(_K)] for k in range(_K)]
    bv = [plsc.load_gather(wb_v, [jnp.full((_LANES,), 26 + k, jnp.int32)])
          for k in range(_K)]
    zeros = jnp.zeros((_LANES,), jnp.float32)
    xpad_v[pl.ds(0, _LANES)] = zeros
    xpad_v[pl.ds(_T, _LANES)] = zeros

    def row_body(r, carry):
        row = wid * _ROWS_PER + r
        pltpu.sync_copy(x_hbm.at[row], xpad_v.at[pl.ds(_PAD, _T)])

        def t_body(i, carry2):
            t0 = i * _LANES
            tvec = iota + t0
            tf = iotaf + t0.astype(jnp.float32)
            s = [_round_bf16(plsc.load_gather(xpad_v, [tvec + (j + _PAD - 2)]))
                 for j in range(_K)]
            acc = zeros
            for k in range(_K):
                z = s[0] * wv[k][0]
                for j in range(1, _K):
                    z = z + s[j] * wv[k][j]
                z = z + bv[k]
                zc = jnp.clip(z, -20.0, 20.0)
                e = jnp.exp(zc + zc)
                den = e + 1.0
                q = 1.0 / den
                q = q * (2.0 - den * q)
                off = (e - 1.0) * q
                pos = off * 5.0 + tf
                pos = pos + float(k - 2)
                pos = pos + 7.0
                il = pos.astype(jnp.int32)
                idx = il - 7
                idx = idx + jnp.where(idx < 0, 256, 0)
                idx = idx - jnp.where(idx > 255, 256, 0)
                acc = acc + plsc.load_gather(xpad_v, [idx + _PAD])
            plsc.store_scatter(out_v, [tvec], acc)
            return carry2

        lax.fori_loop(0, _T // _LANES, t_body, 0)
        pltpu.sync_copy(out_v, out_hbm.at[row])
        return carry

    lax.fori_loop(0, _ROWS_PER, row_body, 0)


@jax.jit
def kernel(x, W, b):
    B, F, C, T = x.shape
    x2 = x.reshape(B * C, T)
    wb = jnp.concatenate(
        [jnp.zeros((1,), jnp.float32), W.reshape(-1), b.reshape(-1),
         jnp.zeros((1,), jnp.float32)])
    mesh = plsc.VectorSubcoreMesh(core_axis_name="c", subcore_axis_name="s")
    run = functools.partial(
        pl.kernel,
        out_type=jax.ShapeDtypeStruct((B * C, T), jnp.float32),
        scratch_types=[
            pltpu.VMEM((_XPAD,), jnp.float32),
            pltpu.VMEM((_T,), jnp.float32),
            pltpu.VMEM((32,), jnp.float32),
        ],
        mesh=mesh,
        compiler_params=pltpu.CompilerParams(
            needs_layout_passes=False, use_tc_tiling_on_sc=False),
    )(_sc_body)
    out = run(x2, wb)
    return out.reshape(B, 1, C, T)


# speedup vs baseline: 121.6965x; 1.0732x over previous
import functools

import jax
import jax.numpy as jnp
from jax import lax
from jax.experimental import pallas as pl
from jax.experimental.pallas import tpu as pltpu
from jax.experimental.pallas import tpu_sc as plsc

_K = 5
_T = 256
_NROWS = 256
_NC = 2
_NS = 16
_NW = _NC * _NS
_RPW = _NROWS // _NW
_RSTR = 272
_PAD = 8
_LANES = 16


def _round_bf16(v):
    u = plsc.bitcast(v, jnp.int32)
    lsb = jax.lax.shift_right_logical(u, 16) & 1
    u = (u + 0x7FFF + lsb) & jnp.int32(-65536)
    return plsc.bitcast(u, jnp.float32)


def _sc_body(x_hbm, wb_hbm, out_hbm, xall_v, xr_v, out_v, wb_v):
    wid = lax.axis_index("s") * _NC + lax.axis_index("c")
    base = wid * (_RPW * _T)
    pltpu.sync_copy(wb_hbm, wb_v)
    pltpu.sync_copy(x_hbm.at[pl.ds(base, _RPW * _T)], xall_v)
    iota = lax.iota(jnp.int32, _LANES)
    iotaf = iota.astype(jnp.float32)
    wv = [[_round_bf16(
        plsc.load_gather(wb_v, [jnp.full((_LANES,), 1 + k * _K + j, jnp.int32)]))
           for j in range(_K)] for k in range(_K)]
    bv = [plsc.load_gather(wb_v, [jnp.full((_LANES,), 26 + k, jnp.int32)])
          for k in range(_K)]
    zeros = jnp.zeros((_LANES,), jnp.float32)

    for r in range(_RPW):
        xr_v[pl.ds(r * _RSTR, _LANES)] = zeros
        xr_v[pl.ds(r * _RSTR + _T, _LANES)] = zeros

        def round_body(i, c, r=r):
            v = xall_v[pl.ds(r * _T + i * _LANES, _LANES)]
            xr_v[pl.ds(r * _RSTR + _PAD + i * _LANES, _LANES)] = _round_bf16(v)
            return c

        lax.fori_loop(0, _T // _LANES, round_body, 0)

    for r in range(_RPW):
        rbase = r * _T
        xrbase = r * _RSTR + _PAD

        def t_body(i, c, rbase=rbase, xrbase=xrbase):
            t0 = i * _LANES
            tvec = iota + t0
            tf = iotaf + t0.astype(jnp.float32)
            s = [xr_v[pl.ds(xrbase + t0 + (j - 2), _LANES)] for j in range(_K)]
            acc = zeros
            for k in range(_K):
                z = s[0] * wv[k][0]
                for j in range(1, _K):
                    z = z + s[j] * wv[k][j]
                z = z + bv[k]
                zc = jnp.clip(z, -20.0, 20.0)
                e = jnp.exp(zc + zc)
                off = (e - 1.0) / (e + 1.0)
                pos = off * 5.0 + tf
                pos = pos + float(k - 2)
                pos = pos + 7.0
                il = pos.astype(jnp.int32)
                idx = ((il + 249) & 255) + rbase
                acc = acc + plsc.load_gather(xall_v, [idx])
            out_v[pl.ds(rbase + t0, _LANES)] = acc
            return c

        lax.fori_loop(0, _T // _LANES, t_body, 0)

    pltpu.sync_copy(out_v, out_hbm.at[pl.ds(base, _RPW * _T)])


@jax.jit
def kernel(x, W, b):
    B, F, C, T = x.shape
    x2 = x.reshape(B * C * T)
    wb = jnp.concatenate(
        [jnp.zeros((1,), jnp.float32), W.reshape(-1), b.reshape(-1),
         jnp.zeros((1,), jnp.float32)])
    mesh = plsc.VectorSubcoreMesh(core_axis_name="c", subcore_axis_name="s")
    run = functools.partial(
        pl.kernel,
        out_type=jax.ShapeDtypeStruct((B * C * T,), jnp.float32),
        scratch_types=[
            pltpu.VMEM((_RPW * _T,), jnp.float32),
            pltpu.VMEM((_RPW * _RSTR,), jnp.float32),
            pltpu.VMEM((_RPW * _T,), jnp.float32),
            pltpu.VMEM((32,), jnp.float32),
        ],
        mesh=mesh,
        compiler_params=pltpu.CompilerParams(
            needs_layout_passes=False, use_tc_tiling_on_sc=False),
    )(_sc_body)
    out = run(x2, wb)
    return out.reshape(B, 1, C, T)


# speedup vs baseline: 145.3755x; 1.1946x over previous
import functools

import jax
import jax.numpy as jnp
from jax import lax
from jax.experimental import pallas as pl
from jax.experimental.pallas import tpu as pltpu
from jax.experimental.pallas import tpu_sc as plsc

_K = 5
_T = 256
_NROWS = 256
_NC = 2
_NS = 16
_NW = _NC * _NS
_RPW = _NROWS // _NW
_RSTR = 272
_PAD = 8
_LANES = 16


def _round_bf16(v):
    u = plsc.bitcast(v, jnp.int32)
    lsb = jax.lax.shift_right_logical(u, 16) & 1
    u = (u + 0x7FFF + lsb) & jnp.int32(-65536)
    return plsc.bitcast(u, jnp.float32)


def _sc_body(x_hbm, wb_hbm, out_hbm, xall_v, xr_v, out_v, wb_v):
    wid = lax.axis_index("s") * _NC + lax.axis_index("c")
    base = wid * (_RPW * _T)
    pltpu.sync_copy(wb_hbm, wb_v)
    pltpu.sync_copy(x_hbm.at[pl.ds(base, _RPW * _T)], xall_v)
    iota = lax.iota(jnp.int32, _LANES)
    iotaf = iota.astype(jnp.float32)
    wv = [[_round_bf16(
        plsc.load_gather(wb_v, [jnp.full((_LANES,), 1 + k * _K + j, jnp.int32)]))
           for j in range(_K)] for k in range(_K)]
    bv = [plsc.load_gather(wb_v, [jnp.full((_LANES,), 26 + k, jnp.int32)])
          for k in range(_K)]
    zeros = jnp.zeros((_LANES,), jnp.float32)

    for r in range(_RPW):
        xr_v[pl.ds(r * _RSTR, _LANES)] = zeros
        xr_v[pl.ds(r * _RSTR + _T, _LANES)] = zeros

    @plsc.parallel_loop(0, _RPW * (_T // _LANES), unroll=4)
    def _round(i):
        r = jax.lax.shift_right_logical(i, 4)
        i16 = (i & 15) * _LANES
        v = xall_v[pl.ds(r * _T + i16, _LANES)]
        xr_v[pl.ds(r * _RSTR + _PAD + i16, _LANES)] = _round_bf16(v)

    @plsc.parallel_loop(0, _RPW * (_T // _LANES), unroll=2)
    def _main(i):
        r = jax.lax.shift_right_logical(i, 4)
        t0 = (i & 15) * _LANES
        rbase = r * _T
        xrbase = r * _RSTR + _PAD
        tvec = iota + t0
        tf = iotaf + t0.astype(jnp.float32)
        s = [xr_v[pl.ds(xrbase + t0 + (j - 2), _LANES)] for j in range(_K)]
        acc = zeros
        for k in range(_K):
            z = s[0] * wv[k][0]
            for j in range(1, _K):
                z = z + s[j] * wv[k][j]
            z = z + bv[k]
            zc = jnp.clip(z, -20.0, 20.0)
            e = jnp.exp(zc + zc)
            off = (e - 1.0) / (e + 1.0)
            pos = off * 5.0 + tf
            pos = pos + float(k - 2)
            pos = pos + 7.0
            il = pos.astype(jnp.int32)
            idx = ((il + 249) & 255) + rbase
            acc = acc + plsc.load_gather(xall_v, [idx])
        out_v[pl.ds(rbase + t0, _LANES)] = acc

    pltpu.sync_copy(out_v, out_hbm.at[pl.ds(base, _RPW * _T)])


@jax.jit
def kernel(x, W, b):
    B, F, C, T = x.shape
    x2 = x.reshape(B * C * T)
    wb = jnp.concatenate(
        [jnp.zeros((1,), jnp.float32), W.reshape(-1), b.reshape(-1),
         jnp.zeros((1,), jnp.float32)])
    mesh = plsc.VectorSubcoreMesh(core_axis_name="c", subcore_axis_name="s")
    run = functools.partial(
        pl.kernel,
        out_type=jax.ShapeDtypeStruct((B * C * T,), jnp.float32),
        scratch_types=[
            pltpu.VMEM((_RPW * _T,), jnp.float32),
            pltpu.VMEM((_RPW * _RSTR,), jnp.float32),
            pltpu.VMEM((_RPW * _T,), jnp.float32),
            pltpu.VMEM((32,), jnp.float32),
        ],
        mesh=mesh,
        compiler_params=pltpu.CompilerParams(
            needs_layout_passes=False, use_tc_tiling_on_sc=False),
    )(_sc_body)
    out = run(x2, wb)
    return out.reshape(B, 1, C, T)


# speedup vs baseline: 146.2889x; 1.0063x over previous
import functools

import jax
import jax.numpy as jnp
from jax import lax
from jax.experimental import pallas as pl
from jax.experimental.pallas import tpu as pltpu
from jax.experimental.pallas import tpu_sc as plsc

_K = 5
_T = 256
_NROWS = 256
_NC = 2
_NS = 16
_NW = _NC * _NS
_RPW = _NROWS // _NW
_RSTR = 272
_PAD = 8
_LANES = 16


def _round_bf16(v):
    u = plsc.bitcast(v, jnp.int32)
    lsb = jax.lax.shift_right_logical(u, 16) & 1
    u = (u + 0x7FFF + lsb) & jnp.int32(-65536)
    return plsc.bitcast(u, jnp.float32)


def _sc_body(x_hbm, wb_hbm, out_hbm, xall_v, xr_v, out_v, wb_v):
    wid = lax.axis_index("s") * _NC + lax.axis_index("c")
    base = wid * (_RPW * _T)
    pltpu.sync_copy(wb_hbm, wb_v)
    pltpu.sync_copy(x_hbm.at[pl.ds(base, _RPW * _T)], xall_v)
    iota = lax.iota(jnp.int32, _LANES)
    iotaf = iota.astype(jnp.float32)
    wv = [[_round_bf16(
        plsc.load_gather(wb_v, [jnp.full((_LANES,), 1 + k * _K + j, jnp.int32)]))
           for j in range(_K)] for k in range(_K)]
    bv = [plsc.load_gather(wb_v, [jnp.full((_LANES,), 26 + k, jnp.int32)])
          for k in range(_K)]
    zeros = jnp.zeros((_LANES,), jnp.float32)

    for r in range(_RPW):
        xr_v[pl.ds(r * _RSTR, _LANES)] = zeros
        xr_v[pl.ds(r * _RSTR + _T, _LANES)] = zeros

    @plsc.parallel_loop(0, _RPW * (_T // _LANES), unroll=4)
    def _round(i):
        r = jax.lax.shift_right_logical(i, 4)
        i16 = (i & 15) * _LANES
        v = xall_v[pl.ds(r * _T + i16, _LANES)]
        xr_v[pl.ds(r * _RSTR + _PAD + i16, _LANES)] = _round_bf16(v)

    @plsc.parallel_loop(0, _RPW * (_T // _LANES), unroll=4)
    def _main(i):
        r = jax.lax.shift_right_logical(i, 4)
        t0 = (i & 15) * _LANES
        rbase = r * _T
        xrbase = r * _RSTR + _PAD
        tvec = iota + t0
        tf = iotaf + t0.astype(jnp.float32)
        s = [xr_v[pl.ds(xrbase + t0 + (j - 2), _LANES)] for j in range(_K)]
        acc = zeros
        for k in range(_K):
            z = s[0] * wv[k][0]
            for j in range(1, _K):
                z = z + s[j] * wv[k][j]
            z = z + bv[k]
            zc = jnp.clip(z, -20.0, 20.0)
            e = jnp.exp(zc + zc)
            off = (e - 1.0) / (e + 1.0)
            pos = off * 5.0 + tf
            pos = pos + float(k - 2)
            pos = pos + 7.0
            il = pos.astype(jnp.int32)
            idx = ((il + 249) & 255) + rbase
            acc = acc + plsc.load_gather(xall_v, [idx])
        out_v[pl.ds(rbase + t0, _LANES)] = acc

    pltpu.sync_copy(out_v, out_hbm.at[pl.ds(base, _RPW * _T)])


@jax.jit
def kernel(x, W, b):
    B, F, C, T = x.shape
    x2 = x.reshape(B * C * T)
    wb = jnp.concatenate(
        [jnp.zeros((1,), jnp.float32), W.reshape(-1), b.reshape(-1),
         jnp.zeros((1,), jnp.float32)])
    mesh = plsc.VectorSubcoreMesh(core_axis_name="c", subcore_axis_name="s")
    run = functools.partial(
        pl.kernel,
        out_type=jax.ShapeDtypeStruct((B * C * T,), jnp.float32),
        scratch_types=[
            pltpu.VMEM((_RPW * _T,), jnp.float32),
            pltpu.VMEM((_RPW * _RSTR,), jnp.float32),
            pltpu.VMEM((_RPW * _T,), jnp.float32),
            pltpu.VMEM((32,), jnp.float32),
        ],
        mesh=mesh,
        compiler_params=pltpu.CompilerParams(
            needs_layout_passes=False, use_tc_tiling_on_sc=False),
    )(_sc_body)
    out = run(x2, wb)
    return out.reshape(B, 1, C, T)
